# baseline (speedup 1.0000x reference)
import functools

import jax
import jax.numpy as jnp
from jax import lax
from jax.experimental import pallas as pl
from jax.experimental.pallas import tpu as pltpu
from jax.experimental.pallas import tpu_sc as plsc

_B = 2
_NS = 26
_ND = 13
_NF = _NS + _ND
_V = 100000
_D = 32
_L = 16
_SPT = 4
_NLT = _NS * _B // _SPT
_DT0 = _NLT


def _body(x_hbm, tbl_hbm, dt_hbm, out_hbm, x_v, dt_v, comb_v, dcomb_v,
          stage_v, sem):
    sid = lax.axis_index("s")
    wid = sid

    lanes = lax.iota(jnp.int32, _L)

    @pl.when(wid < _DT0 + _B)
    def _():
        pltpu.sync_copy(x_hbm, x_v)

        @pl.when(wid < _DT0)
        def _():
            pairs = []
            copies = []
            for p in range(_SPT // 2):
                s0 = wid * _SPT + 2 * p
                b = (s0 >= _NS).astype(jnp.int32)
                f0 = s0 - b * _NS

                ids = plsc.load_gather(
                    x_v, [jnp.broadcast_to(b, (_L,)), lanes + f0]
                ).astype(jnp.int32)
                blk = jnp.bitwise_and(ids, -128)
                col = jnp.bitwise_and(ids, 127)
                pairs.append((s0, b, col))

                for k in range(2):
                    off = pl.multiple_of(blk[k], 128)
                    copies.append(pltpu.async_copy(
                        tbl_hbm.at[f0 + k, pl.ds(0, _D), pl.ds(off, 128)],
                        stage_v.at[pl.ds((2 * p + k) * _D, _D)], sem))
            for cp in copies:
                cp.wait()

            outs = []
            for p, (s0, b, col) in enumerate(pairs):
                for k in range(2):
                    i1 = jnp.broadcast_to(col[k], (_L,))
                    for h in range(2):
                        o = (2 * p + k) * _D + h * _L
                        comb_v[pl.ds(o, _L)] = plsc.load_gather(
                            stage_v, [lanes + o, i1])
                outs.append(pltpu.async_copy(
                    comb_v.at[pl.ds(2 * p * _D, 2 * _D)],
                    out_hbm.at[pl.ds(
                        pl.multiple_of(s0 * _D + b * _ND * _D, _D),
                        2 * _D)], sem))
            for cp in outs:
                cp.wait()

        @pl.when(wid >= _DT0)
        def _():
            b = wid - _DT0
            pltpu.sync_copy(dt_hbm, dt_v)
            dv = plsc.load_gather(
                x_v, [jnp.broadcast_to(b, (_L,)), lanes + _NF - _L])
            iz = jnp.broadcast_to(jnp.int32(0), (_L,))
            for j in range(_ND):
                sc = dv[j + _L - _ND]
                ij = jnp.broadcast_to(jnp.int32(j), (_L,))
                for h in range(2):
                    o = j * _D + h * _L
                    dcomb_v[pl.ds(o, _L)] = sc * plsc.load_gather(
                        dt_v, [ij, iz, lanes + h * _L])

            pltpu.sync_copy(
                dcomb_v,
                out_hbm.at[pl.ds(
                    pl.multiple_of((b * _NF + _NS) * _D, _L), _ND * _D)])


_sc_call = functools.partial(
    pl.kernel,
    mesh=plsc.VectorSubcoreMesh(
        core_axis_name="c", subcore_axis_name="s", num_cores=1),
    out_type=jax.ShapeDtypeStruct((_B * _NF * _D,), jnp.float32),
    compiler_params=pltpu.CompilerParams(
        needs_layout_passes=False, skip_device_barrier=True),
    scratch_types=[
        pltpu.VMEM((_B, _NF), jnp.float32),
        pltpu.VMEM((_ND, 1, _D), jnp.float32),
        pltpu.VMEM((_SPT * _D,), jnp.float32),
        pltpu.VMEM((_ND * _D,), jnp.float32),
        pltpu.VMEM((_SPT * _D, 128), jnp.float32),
        pltpu.SemaphoreType.DMA,
    ],
)(_body)


def kernel(X, sparse_tables, dense_tables):
    tbl = sparse_tables.transpose(0, 2, 1)
    out = _sc_call(X, tbl, dense_tables)
    return out.reshape(_B, _NF, _D)


# speedup vs baseline: 1.0017x; 1.0017x over previous
import functools

import jax
import jax.numpy as jnp
from jax import lax
from jax.experimental import pallas as pl
from jax.experimental.pallas import tpu as pltpu
from jax.experimental.pallas import tpu_sc as plsc

_B = 2
_NS = 26
_ND = 13
_NF = _NS + _ND
_V = 100000
_D = 32
_L = 16
_SPT = 4
_NLT = _NS * _B // _SPT
_DT0 = _NLT


def _body(x_hbm, tbl_hbm, dt_hbm, out_hbm, x_v, dt_v, comb_v, dcomb_v,
          stage_v, sem):
    sid = lax.axis_index("s")
    wid = sid

    lanes = lax.iota(jnp.int32, _L)

    @pl.when(wid < _DT0 + _B)
    def _():
        pltpu.sync_copy(x_hbm, x_v)

        @pl.when(wid < _DT0)
        def _():
            pairs = []
            copies = []
            for p in range(_SPT // 2):
                s0 = wid * _SPT + 2 * p
                b = (s0 >= _NS).astype(jnp.int32)
                f0 = s0 - b * _NS

                ids = plsc.load_gather(
                    x_v, [jnp.broadcast_to(b, (_L,)), lanes + f0]
                ).astype(jnp.int32)
                blk = jnp.bitwise_and(ids, -128)
                col = jnp.bitwise_and(ids, 127)
                pairs.append((s0, b, col))

                for k in range(2):
                    off = pl.multiple_of(blk[k], 128)
                    copies.append(pltpu.async_copy(
                        tbl_hbm.at[f0 + k, pl.ds(0, _D), pl.ds(off, 128)],
                        stage_v.at[pl.ds((2 * p + k) * _D, _D)], sem))
            for cp in copies:
                cp.wait()

            outs = []
            for p, (s0, b, col) in enumerate(pairs):
                for k in range(2):
                    i1 = jnp.broadcast_to(col[k], (_L,))
                    for h in range(2):
                        o = (2 * p + k) * _D + h * _L
                        comb_v[pl.ds(o, _L)] = plsc.load_gather(
                            stage_v, [lanes + o, i1])
                outs.append(pltpu.async_copy(
                    comb_v.at[pl.ds(2 * p * _D, 2 * _D)],
                    out_hbm.at[pl.ds(
                        pl.multiple_of(s0 * _D + b * _ND * _D, _D),
                        2 * _D)], sem))
            for cp in outs:
                cp.wait()

        @pl.when(wid >= _DT0)
        def _():
            b = wid - _DT0
            pltpu.sync_copy(dt_hbm, dt_v)
            dv = plsc.load_gather(
                x_v, [jnp.broadcast_to(b, (_L,)), lanes + _NF - _L])
            iz = jnp.broadcast_to(jnp.int32(0), (_L,))
            for j in range(_ND):
                sc = dv[j + _L - _ND]
                ij = jnp.broadcast_to(jnp.int32(j), (_L,))
                for h in range(2):
                    o = j * _D + h * _L
                    dcomb_v[pl.ds(o, _L)] = sc * plsc.load_gather(
                        dt_v, [ij, iz, lanes + h * _L])

            pltpu.sync_copy(
                dcomb_v,
                out_hbm.at[pl.ds(
                    pl.multiple_of((b * _NF + _NS) * _D, _L), _ND * _D)])


_sc_call = functools.partial(
    pl.kernel,
    mesh=plsc.VectorSubcoreMesh(
        core_axis_name="c", subcore_axis_name="s", num_cores=1),
    out_type=jax.ShapeDtypeStruct((_B * _NF * _D,), jnp.float32),
    compiler_params=pltpu.CompilerParams(needs_layout_passes=False),
    scratch_types=[
        pltpu.VMEM((_B, _NF), jnp.float32),
        pltpu.VMEM((_ND, 1, _D), jnp.float32),
        pltpu.VMEM((_SPT * _D,), jnp.float32),
        pltpu.VMEM((_ND * _D,), jnp.float32),
        pltpu.VMEM((_SPT * _D, 128), jnp.float32),
        pltpu.SemaphoreType.DMA,
    ],
)(_body)


def kernel(X, sparse_tables, dense_tables):
    tbl = sparse_tables.transpose(0, 2, 1)
    out = _sc_call(X, tbl, dense_tables)
    return out.reshape(_B, _NF, _D)
